# baseline (speedup 1.0000x reference)
import functools

import jax
import jax.numpy as jnp
from jax import lax
from jax.experimental import pallas as pl
from jax.experimental.pallas import tpu as pltpu
from jax.experimental.pallas import tpu_sc as plsc

N_NODES = 50000
N_EDGES = 800000
HIDDEN = 64
NRBF = 20
CUTOFF = 5.0
NLAYERS = 3
NUM_EMB = 119

NC, NS = 2, 16
NW = NC * NS
E_PAD = 802816
EW = E_PAD // NW
NODES_PER_R = 784
NRANGES = 64
NODE_PAD = NRANGES * NODES_PER_R

_mesh = plsc.VectorSubcoreMesh(core_axis_name="c", subcore_axis_name="s")


def _make_gather(d, chunk):
    assert EW % chunk == 0 and chunk % 128 == 0 and d % 128 == 0
    n_chunks = EW // chunk

    @functools.partial(
        pl.kernel,
        out_type=jax.ShapeDtypeStruct((E_PAD, d), jnp.float32),
        mesh=_mesh,
        scratch_types=[
            pltpu.VMEM((chunk,), jnp.int32),
            pltpu.VMEM((chunk, d), jnp.float32),
            pltpu.SemaphoreType.DMA,
        ],
    )
    def gather_kernel(table_hbm, idx_hbm, out_hbm, idx_v, rows_v, sem):
        w = lax.axis_index("s") * NC + lax.axis_index("c")
        base = w * EW

        def body(ci, carry):
            st = base + ci * chunk
            pltpu.sync_copy(idx_hbm.at[pl.ds(st, chunk)], idx_v)
            pltpu.async_copy(table_hbm.at[idx_v], rows_v, sem).wait()
            pltpu.sync_copy(rows_v, out_hbm.at[pl.ds(st, chunk)])
            return carry

        lax.fori_loop(0, n_chunks, body, 0)

    return gather_kernel


_gather_128 = _make_gather(128, 512)
_gather_256 = _make_gather(256, 256)
_gather_384 = _make_gather(384, 256)


_CH = 128


@functools.partial(
    pl.kernel,
    out_type=jax.ShapeDtypeStruct((NODE_PAD, 256), jnp.float32),
    mesh=_mesh,
    scratch_types=[
        pltpu.VMEM((NODES_PER_R + 1, 128), jnp.float32),
        pltpu.VMEM((_CH,), jnp.int32),
        pltpu.VMEM((_CH, 128), jnp.float32),
        pltpu.VMEM((16,), jnp.int32),
    ],
)
def _segsum_kernel(rows_hbm, dst_hbm, rb_hbm, out_hbm, acc, dstb, rowb, rbv):
    w = lax.axis_index("s") * NC + lax.axis_index("c")
    iota16 = lax.broadcasted_iota(jnp.int32, (16,), 0)

    for half in range(2):
        r = w + half * NW
        n0 = r * NODES_PER_R
        pltpu.sync_copy(rb_hbm.at[pl.ds(r * 16, 16)], rbv)
        bounds = rbv[...]
        e0 = bounds[0]
        e1 = bounds[1]
        e0a = (e0 // _CH) * _CH
        n_chunks = (e1 - e0a + (_CH - 1)) // _CH

        for slab in range(2):
            def zero_body(i, carry):
                for t in range(8):
                    acc[i, pl.ds(16 * t, 16)] = jnp.zeros((16,), jnp.float32)
                return carry

            lax.fori_loop(0, NODES_PER_R, zero_body, 0)

            def chunk_body(ci, carry):
                st = e0a + ci * _CH
                pltpu.sync_copy(dst_hbm.at[pl.ds(st, _CH)], dstb)
                pltpu.sync_copy(
                    rows_hbm.at[pl.ds(st, _CH), pl.ds(128 * slab, 128)], rowb
                )

                def group_body(g, gc):
                    cur = gc[0]
                    vs = list(gc[1:])
                    localv = dstb[pl.ds(16 * g, 16)] - n0
                    for j in range(16):
                        lv = localv[j]
                        d = (jnp.broadcast_to(lv, (16,))
                             - jnp.broadcast_to(cur, (16,)))
                        kf = (1 - jnp.minimum(jnp.abs(d), 1)).astype(jnp.float32)
                        tgt = jnp.where((lv >= 0) & (lv < NODES_PER_R),
                                        lv, NODES_PER_R)
                        for t in range(8):
                            row = rowb[16 * g + j, pl.ds(16 * t, 16)]
                            vs[t] = vs[t] * kf + row
                            acc[tgt, pl.ds(16 * t, 16)] = vs[t]
                        cur = lv
                    return (cur,) + tuple(vs)

                return lax.fori_loop(0, _CH // 16, group_body, carry)

            zero16 = jnp.zeros((16,), jnp.float32)
            init = (jnp.int32(-1),) + (zero16,) * 8
            lax.fori_loop(0, n_chunks, chunk_body, init)
            pltpu.sync_copy(
                acc.at[pl.ds(0, NODES_PER_R)],
                out_hbm.at[pl.ds(n0, NODES_PER_R), pl.ds(128 * slab, 128)],
            )


_BE = 512
_BN = 1000


def _geom_body(gps_ref, gpd_ref, geom_ref):
    diff = gpd_ref[...][:, 0:3] - gps_ref[...][:, 0:3]
    dist = jnp.sqrt(jnp.sum(diff * diff, axis=1, keepdims=True) + 1e-12)
    n_arr = lax.broadcasted_iota(jnp.int32, (1, NRBF), 1).astype(jnp.float32) + 1.0
    es = jnp.sin(n_arr * (jnp.pi / CUTOFF) * dist) / dist
    fc = jnp.where(dist < CUTOFF, 0.5 * (jnp.cos(dist * (jnp.pi / CUTOFF)) + 1.0), 0.0)
    unit = diff / dist
    geom_ref[...] = jnp.concatenate([es, fc, unit], axis=1)


def _geom(gps, gpd):
    return pl.pallas_call(
        _geom_body,
        grid=(E_PAD // _BE,),
        in_specs=[
            pl.BlockSpec((_BE, 128), lambda i: (i, 0)),
            pl.BlockSpec((_BE, 128), lambda i: (i, 0)),
        ],
        out_specs=pl.BlockSpec((_BE, NRBF + 4), lambda i: (i, 0)),
        out_shape=jax.ShapeDtypeStruct((E_PAD, NRBF + 4), jnp.float32),
    )(gps, gpd)


def _embed_smlp_body(xa_ref, emb_ref, w1_ref, b1_ref, w2_ref, b2_ref, ns_ref, s_ref):
    xa = xa_ref[...]
    ids = lax.broadcasted_iota(jnp.int32, (_BN, NUM_EMB), 1)
    onehot = (ids == xa).astype(jnp.float32)
    ns = jnp.dot(onehot, emb_ref[...], preferred_element_type=jnp.float32)
    h = jnp.dot(ns, w1_ref[...], preferred_element_type=jnp.float32) + b1_ref[...]
    h = h * jax.nn.sigmoid(h)
    s = jnp.dot(h, w2_ref[...], preferred_element_type=jnp.float32) + b2_ref[...]
    ns_ref[...] = ns
    s_ref[...] = jnp.concatenate([s, jnp.zeros((_BN, 64), jnp.float32)], axis=1)


def _embed_smlp(x_atoms2, emb, w1, b1, w2, b2):
    return pl.pallas_call(
        _embed_smlp_body,
        grid=(N_NODES // _BN,),
        in_specs=[
            pl.BlockSpec((_BN, 1), lambda i: (i, 0)),
            pl.BlockSpec(emb.shape, lambda i: (0, 0)),
            pl.BlockSpec((HIDDEN, HIDDEN), lambda i: (0, 0)),
            pl.BlockSpec((1, HIDDEN), lambda i: (0, 0)),
            pl.BlockSpec((HIDDEN, 3 * HIDDEN), lambda i: (0, 0)),
            pl.BlockSpec((1, 3 * HIDDEN), lambda i: (0, 0)),
        ],
        out_specs=[
            pl.BlockSpec((_BN, HIDDEN), lambda i: (i, 0)),
            pl.BlockSpec((_BN, 256), lambda i: (i, 0)),
        ],
        out_shape=[
            jax.ShapeDtypeStruct((N_NODES, HIDDEN), jnp.float32),
            jax.ShapeDtypeStruct((N_NODES, 256), jnp.float32),
        ],
    )(x_atoms2, emb, w1, b1, w2, b2)


def _make_msg(has_nv):
    def body(geom_ref, g_ref, fw_ref, fb_ref, rows_ref):
        geom = geom_ref[...]
        es = geom[:, 0:NRBF]
        fcv = geom[:, NRBF:NRBF + 1]
        unit = geom[:, NRBF + 1:NRBF + 4]
        gtab = g_ref[...]
        filt = (jnp.dot(es, fw_ref[...], preferred_element_type=jnp.float32)
                + fb_ref[...]) * fcv
        msg = filt * gtab[:, 0:3 * HIDDEN]
        gate_sv = msg[:, 0:HIDDEN]
        gate_ev = msg[:, HIDDEN:2 * HIDDEN]
        msg_s = msg[:, 2 * HIDDEN:3 * HIDDEN]
        parts = [msg_s]
        for dd in range(3):
            mv_d = gate_ev * unit[:, dd:dd + 1]
            if has_nv:
                mv_d = mv_d + gtab[:, (3 + dd) * HIDDEN:(4 + dd) * HIDDEN] * gate_sv
            parts.append(mv_d)
        rows_ref[...] = jnp.concatenate(parts, axis=1)

    gw = 384 if has_nv else 256

    def call(geom, gtab, fw, fb):
        return pl.pallas_call(
            body,
            grid=(E_PAD // _BE,),
            in_specs=[
                pl.BlockSpec((_BE, NRBF + 4), lambda i: (i, 0)),
                pl.BlockSpec((_BE, gw), lambda i: (i, 0)),
                pl.BlockSpec((NRBF, 3 * HIDDEN), lambda i: (0, 0)),
                pl.BlockSpec((1, 3 * HIDDEN), lambda i: (0, 0)),
            ],
            out_specs=pl.BlockSpec((_BE, 4 * HIDDEN), lambda i: (i, 0)),
            out_shape=jax.ShapeDtypeStruct((E_PAD, 4 * HIDDEN), jnp.float32),
        )(geom, gtab, fw, fb)

    return call


_msg_l0 = _make_msg(False)
_msg = _make_msg(True)


def _make_update(has_nv, with_s):
    def body(*refs):
        i = 0
        ns_ref = refs[i]; i += 1
        if has_nv:
            nv_ref = refs[i]; i += 1
        seg_ref = refs[i]; i += 1
        u_ref = refs[i]; i += 1
        v_ref = refs[i]; i += 1
        w1_ref = refs[i]; i += 1
        b1_ref = refs[i]; i += 1
        w2_ref = refs[i]; i += 1
        b2_ref = refs[i]; i += 1
        if with_s:
            mw1_ref = refs[i]; i += 1
            mb1_ref = refs[i]; i += 1
            mw2_ref = refs[i]; i += 1
            mb2_ref = refs[i]; i += 1
        nso_ref = refs[i]; i += 1
        nvo_ref = refs[i]; i += 1

        seg = seg_ref[...]
        ns1 = ns_ref[...] + seg[:, 0:HIDDEN]
        nv1 = seg[:, HIDDEN:4 * HIDDEN]
        if has_nv:
            nv1 = nv1 + nv_ref[...][:, 3 * HIDDEN:6 * HIDDEN]
        u = u_ref[...]
        v = v_ref[...]
        uv = []
        vv = []
        for dd in range(3):
            nv_d = nv1[:, dd * HIDDEN:(dd + 1) * HIDDEN]
            uv.append(jnp.dot(nv_d, u, preferred_element_type=jnp.float32))
            vv.append(jnp.dot(nv_d, v, preferred_element_type=jnp.float32))
        vn = jnp.sqrt(vv[0] * vv[0] + vv[1] * vv[1] + vv[2] * vv[2] + 1e-12)
        h = jnp.concatenate([ns1, vn], axis=1)
        h = jnp.dot(h, w1_ref[...], preferred_element_type=jnp.float32) + b1_ref[...]
        h = h * jax.nn.sigmoid(h)
        a = jnp.dot(h, w2_ref[...], preferred_element_type=jnp.float32) + b2_ref[...]
        a_vv = a[:, 0:HIDDEN]
        a_sv = a[:, HIDDEN:2 * HIDDEN]
        a_ss = a[:, 2 * HIDDEN:3 * HIDDEN]
        dotuv = uv[0] * vv[0] + uv[1] * vv[1] + uv[2] * vv[2]
        ns2 = ns1 + dotuv * a_sv + a_ss
        nv2 = jnp.concatenate([nv1[:, dd * HIDDEN:(dd + 1) * HIDDEN]
                               + uv[dd] * a_vv for dd in range(3)], axis=1)
        nso_ref[...] = ns2
        if with_s:
            hs = jnp.dot(ns2, mw1_ref[...], preferred_element_type=jnp.float32) + mb1_ref[...]
            hs = hs * jax.nn.sigmoid(hs)
            s_next = jnp.dot(hs, mw2_ref[...], preferred_element_type=jnp.float32) + mb2_ref[...]
            nvo_ref[...] = jnp.concatenate([s_next, nv2], axis=1)
        else:
            nvo_ref[...] = nv2

    def call(ns, nv, seg, u, v, w1, b1, w2, b2, msg_w1=None, msg_b1=None,
             msg_w2=None, msg_b2=None):
        in_specs = [pl.BlockSpec((_BN, HIDDEN), lambda i: (i, 0))]
        args = [ns]
        if has_nv:
            in_specs.append(pl.BlockSpec((_BN, 6 * HIDDEN), lambda i: (i, 0)))
            args.append(nv)
        in_specs.append(pl.BlockSpec((_BN, 4 * HIDDEN), lambda i: (i, 0)))
        args.append(seg)
        for wref in (u, v, w1, b1, w2, b2):
            in_specs.append(pl.BlockSpec(wref.shape, lambda i: (0, 0)))
            args.append(wref)
        if with_s:
            for wref in (msg_w1, msg_b1, msg_w2, msg_b2):
                in_specs.append(pl.BlockSpec(wref.shape, lambda i: (0, 0)))
                args.append(wref)
        owidth = 6 * HIDDEN if with_s else 3 * HIDDEN
        out_specs = [
            pl.BlockSpec((_BN, HIDDEN), lambda i: (i, 0)),
            pl.BlockSpec((_BN, owidth), lambda i: (i, 0)),
        ]
        out_shape = [
            jax.ShapeDtypeStruct((N_NODES, HIDDEN), jnp.float32),
            jax.ShapeDtypeStruct((N_NODES, owidth), jnp.float32),
        ]
        return pl.pallas_call(
            body,
            grid=(N_NODES // _BN,),
            in_specs=in_specs,
            out_specs=out_specs,
            out_shape=out_shape,
        )(*args)

    return call


_update_l0 = _make_update(False, True)
_update_mid = _make_update(True, True)
_update_last = _make_update(True, False)


def kernel(x_atoms, x_pos, edges, embed_table, filter_W, filter_b, msg_W1, msg_b1,
           msg_W2, msg_b2, U_W, V_W, upd_W1, upd_b1, upd_W2, upd_b2):
    src = edges[:, 0].astype(jnp.int32)
    dst = edges[:, 1].astype(jnp.int32)
    perm = jnp.argsort(dst)
    pad = E_PAD - N_EDGES
    src_p = jnp.concatenate([src[perm], jnp.arange(pad, dtype=jnp.int32) % N_NODES])
    dst_p = jnp.concatenate([dst[perm], jnp.full((pad,), N_NODES, jnp.int32)])
    n0s = jnp.arange(NRANGES, dtype=jnp.int32) * NODES_PER_R
    starts = jnp.searchsorted(dst_p, n0s).astype(jnp.int32)
    ends = jnp.searchsorted(dst_p, n0s + NODES_PER_R).astype(jnp.int32)
    rb = jnp.zeros((NRANGES, 16), jnp.int32)
    rb = rb.at[:, 0].set(starts).at[:, 1].set(ends).reshape(-1)
    xp128 = jnp.pad(x_pos.astype(jnp.float32), ((0, 0), (0, 125)))
    x_atoms2 = x_atoms.astype(jnp.int32).reshape(N_NODES, 1)

    fb2 = filter_b.reshape(NLAYERS, 1, 3 * HIDDEN)
    mb1 = msg_b1.reshape(NLAYERS, 1, HIDDEN)
    mb2 = msg_b2.reshape(NLAYERS, 1, 3 * HIDDEN)
    ub1 = upd_b1.reshape(NLAYERS, 1, HIDDEN)
    ub2 = upd_b2.reshape(NLAYERS, 1, 3 * HIDDEN)

    gps = _gather_128(xp128, src_p)
    gpd = _gather_128(xp128, jnp.minimum(dst_p, N_NODES - 1))
    geom = _geom(gps, gpd)

    ns, s = _embed_smlp(x_atoms2, embed_table, msg_W1[0], mb1[0], msg_W2[0], mb2[0])
    gt = _gather_256(s, src_p)
    rows = _msg_l0(geom, gt, filter_W[0], fb2[0])
    seg = _segsum_kernel(rows, dst_p, rb)
    ns, t = _update_l0(ns, None, seg, U_W[0], V_W[0], upd_W1[0], ub1[0],
                       upd_W2[0], ub2[0], msg_W1[1], mb1[1], msg_W2[1], mb2[1])

    gt = _gather_384(t, src_p)
    rows = _msg(geom, gt, filter_W[1], fb2[1])
    seg = _segsum_kernel(rows, dst_p, rb)
    ns, t = _update_mid(ns, t, seg, U_W[1], V_W[1], upd_W1[1], ub1[1],
                        upd_W2[1], ub2[1], msg_W1[2], mb1[2], msg_W2[2], mb2[2])

    gt = _gather_384(t, src_p)
    rows = _msg(geom, gt, filter_W[2], fb2[2])
    seg = _segsum_kernel(rows, dst_p, rb)
    ns, nv = _update_last(ns, t, seg, U_W[2], V_W[2], upd_W1[2], ub1[2],
                          upd_W2[2], ub2[2])

    return ns, nv.reshape(N_NODES, 3, HIDDEN)


# speedup vs baseline: 1.0189x; 1.0189x over previous
import functools

import jax
import jax.numpy as jnp
from jax import lax
from jax.experimental import pallas as pl
from jax.experimental.pallas import tpu as pltpu
from jax.experimental.pallas import tpu_sc as plsc

N_NODES = 50000
N_EDGES = 800000
HIDDEN = 64
NRBF = 20
CUTOFF = 5.0
NLAYERS = 3
NUM_EMB = 119

NC, NS = 2, 16
NW = NC * NS
E_PAD = 802816
EW = E_PAD // NW
NODES_PER_R = 528
NRANGES = 96
NODE_PAD = NRANGES * NODES_PER_R

_mesh = plsc.VectorSubcoreMesh(core_axis_name="c", subcore_axis_name="s")


def _make_gather(d, chunk):
    assert EW % chunk == 0 and chunk % 128 == 0 and d % 128 == 0
    n_chunks = EW // chunk

    @functools.partial(
        pl.kernel,
        out_type=jax.ShapeDtypeStruct((E_PAD, d), jnp.float32),
        mesh=_mesh,
        scratch_types=[
            pltpu.VMEM((chunk,), jnp.int32),
            pltpu.VMEM((chunk, d), jnp.float32),
            pltpu.SemaphoreType.DMA,
        ],
    )
    def gather_kernel(table_hbm, idx_hbm, out_hbm, idx_v, rows_v, sem):
        w = lax.axis_index("s") * NC + lax.axis_index("c")
        base = w * EW

        def body(ci, carry):
            st = base + ci * chunk
            pltpu.sync_copy(idx_hbm.at[pl.ds(st, chunk)], idx_v)
            pltpu.async_copy(table_hbm.at[idx_v], rows_v, sem).wait()
            pltpu.sync_copy(rows_v, out_hbm.at[pl.ds(st, chunk)])
            return carry

        lax.fori_loop(0, n_chunks, body, 0)

    return gather_kernel


_gather_128 = _make_gather(128, 512)
_gather_256 = _make_gather(256, 256)
_gather_384 = _make_gather(384, 256)


_CH = 256


@functools.partial(
    pl.kernel,
    out_type=jax.ShapeDtypeStruct((NODE_PAD, 256), jnp.float32),
    mesh=_mesh,
    scratch_types=[
        pltpu.VMEM((NODES_PER_R + 1, 128), jnp.float32),
        pltpu.VMEM((_CH,), jnp.int32),
        pltpu.VMEM((_CH, 128), jnp.float32),
        pltpu.VMEM((16,), jnp.int32),
    ],
)
def _segsum_kernel(rows_hbm, dst_hbm, rb_hbm, out_hbm, acc, dstb, rowb, rbv):
    w = lax.axis_index("s") * NC + lax.axis_index("c")
    iota16 = lax.broadcasted_iota(jnp.int32, (16,), 0)

    for half in range(NRANGES // NW):
        r = w + half * NW
        n0 = r * NODES_PER_R
        pltpu.sync_copy(rb_hbm.at[pl.ds(r * 16, 16)], rbv)
        bounds = rbv[...]
        e0 = bounds[0]
        e1 = bounds[1]
        e0a = (e0 // _CH) * _CH
        n_chunks = (e1 - e0a + (_CH - 1)) // _CH

        for slab in range(2):
            def zero_body(i, carry):
                for t in range(8):
                    acc[i, pl.ds(16 * t, 16)] = jnp.zeros((16,), jnp.float32)
                return carry

            lax.fori_loop(0, NODES_PER_R, zero_body, 0)

            def chunk_body(ci, carry):
                st = e0a + ci * _CH
                pltpu.sync_copy(dst_hbm.at[pl.ds(st, _CH)], dstb)
                pltpu.sync_copy(
                    rows_hbm.at[pl.ds(st, _CH), pl.ds(128 * slab, 128)], rowb
                )

                def group_body(g, gc):
                    cur = gc[0]
                    vs = list(gc[1:])
                    localv = dstb[pl.ds(16 * g, 16)] - n0
                    for j in range(16):
                        lv = localv[j]
                        d = (jnp.broadcast_to(lv, (16,))
                             - jnp.broadcast_to(cur, (16,)))
                        kf = (1 - jnp.minimum(jnp.abs(d), 1)).astype(jnp.float32)
                        tgt = jnp.where((lv >= 0) & (lv < NODES_PER_R),
                                        lv, NODES_PER_R)
                        for t in range(8):
                            row = rowb[16 * g + j, pl.ds(16 * t, 16)]
                            vs[t] = vs[t] * kf + row
                            acc[tgt, pl.ds(16 * t, 16)] = vs[t]
                        cur = lv
                    return (cur,) + tuple(vs)

                return lax.fori_loop(0, _CH // 16, group_body, carry)

            zero16 = jnp.zeros((16,), jnp.float32)
            init = (jnp.int32(-1),) + (zero16,) * 8
            lax.fori_loop(0, n_chunks, chunk_body, init)
            pltpu.sync_copy(
                acc.at[pl.ds(0, NODES_PER_R)],
                out_hbm.at[pl.ds(n0, NODES_PER_R), pl.ds(128 * slab, 128)],
            )


_BE = 512
_BN = 1000


def _geom_body(gps_ref, gpd_ref, geom_ref):
    diff = gpd_ref[...][:, 0:3] - gps_ref[...][:, 0:3]
    dist = jnp.sqrt(jnp.sum(diff * diff, axis=1, keepdims=True) + 1e-12)
    n_arr = lax.broadcasted_iota(jnp.int32, (1, NRBF), 1).astype(jnp.float32) + 1.0
    es = jnp.sin(n_arr * (jnp.pi / CUTOFF) * dist) / dist
    fc = jnp.where(dist < CUTOFF, 0.5 * (jnp.cos(dist * (jnp.pi / CUTOFF)) + 1.0), 0.0)
    unit = diff / dist
    geom_ref[...] = jnp.concatenate([es, fc, unit], axis=1)


def _geom(gps, gpd):
    return pl.pallas_call(
        _geom_body,
        grid=(E_PAD // _BE,),
        in_specs=[
            pl.BlockSpec((_BE, 128), lambda i: (i, 0)),
            pl.BlockSpec((_BE, 128), lambda i: (i, 0)),
        ],
        out_specs=pl.BlockSpec((_BE, NRBF + 4), lambda i: (i, 0)),
        out_shape=jax.ShapeDtypeStruct((E_PAD, NRBF + 4), jnp.float32),
    )(gps, gpd)


def _embed_smlp_body(xa_ref, emb_ref, w1_ref, b1_ref, w2_ref, b2_ref, ns_ref, s_ref):
    xa = xa_ref[...]
    ids = lax.broadcasted_iota(jnp.int32, (_BN, NUM_EMB), 1)
    onehot = (ids == xa).astype(jnp.float32)
    ns = jnp.dot(onehot, emb_ref[...], preferred_element_type=jnp.float32)
    h = jnp.dot(ns, w1_ref[...], preferred_element_type=jnp.float32) + b1_ref[...]
    h = h * jax.nn.sigmoid(h)
    s = jnp.dot(h, w2_ref[...], preferred_element_type=jnp.float32) + b2_ref[...]
    ns_ref[...] = ns
    s_ref[...] = jnp.concatenate([s, jnp.zeros((_BN, 64), jnp.float32)], axis=1)


def _embed_smlp(x_atoms2, emb, w1, b1, w2, b2):
    return pl.pallas_call(
        _embed_smlp_body,
        grid=(N_NODES // _BN,),
        in_specs=[
            pl.BlockSpec((_BN, 1), lambda i: (i, 0)),
            pl.BlockSpec(emb.shape, lambda i: (0, 0)),
            pl.BlockSpec((HIDDEN, HIDDEN), lambda i: (0, 0)),
            pl.BlockSpec((1, HIDDEN), lambda i: (0, 0)),
            pl.BlockSpec((HIDDEN, 3 * HIDDEN), lambda i: (0, 0)),
            pl.BlockSpec((1, 3 * HIDDEN), lambda i: (0, 0)),
        ],
        out_specs=[
            pl.BlockSpec((_BN, HIDDEN), lambda i: (i, 0)),
            pl.BlockSpec((_BN, 256), lambda i: (i, 0)),
        ],
        out_shape=[
            jax.ShapeDtypeStruct((N_NODES, HIDDEN), jnp.float32),
            jax.ShapeDtypeStruct((N_NODES, 256), jnp.float32),
        ],
    )(x_atoms2, emb, w1, b1, w2, b2)


def _make_msg(has_nv):
    def body(geom_ref, g_ref, fw_ref, fb_ref, rows_ref):
        geom = geom_ref[...]
        es = geom[:, 0:NRBF]
        fcv = geom[:, NRBF:NRBF + 1]
        unit = geom[:, NRBF + 1:NRBF + 4]
        gtab = g_ref[...]
        filt = (jnp.dot(es, fw_ref[...], preferred_element_type=jnp.float32)
                + fb_ref[...]) * fcv
        msg = filt * gtab[:, 0:3 * HIDDEN]
        gate_sv = msg[:, 0:HIDDEN]
        gate_ev = msg[:, HIDDEN:2 * HIDDEN]
        msg_s = msg[:, 2 * HIDDEN:3 * HIDDEN]
        parts = [msg_s]
        for dd in range(3):
            mv_d = gate_ev * unit[:, dd:dd + 1]
            if has_nv:
                mv_d = mv_d + gtab[:, (3 + dd) * HIDDEN:(4 + dd) * HIDDEN] * gate_sv
            parts.append(mv_d)
        rows_ref[...] = jnp.concatenate(parts, axis=1)

    gw = 384 if has_nv else 256

    def call(geom, gtab, fw, fb):
        return pl.pallas_call(
            body,
            grid=(E_PAD // _BE,),
            in_specs=[
                pl.BlockSpec((_BE, NRBF + 4), lambda i: (i, 0)),
                pl.BlockSpec((_BE, gw), lambda i: (i, 0)),
                pl.BlockSpec((NRBF, 3 * HIDDEN), lambda i: (0, 0)),
                pl.BlockSpec((1, 3 * HIDDEN), lambda i: (0, 0)),
            ],
            out_specs=pl.BlockSpec((_BE, 4 * HIDDEN), lambda i: (i, 0)),
            out_shape=jax.ShapeDtypeStruct((E_PAD, 4 * HIDDEN), jnp.float32),
        )(geom, gtab, fw, fb)

    return call


_msg_l0 = _make_msg(False)
_msg = _make_msg(True)


def _make_update(has_nv, with_s):
    def body(*refs):
        i = 0
        ns_ref = refs[i]; i += 1
        if has_nv:
            nv_ref = refs[i]; i += 1
        seg_ref = refs[i]; i += 1
        u_ref = refs[i]; i += 1
        v_ref = refs[i]; i += 1
        w1_ref = refs[i]; i += 1
        b1_ref = refs[i]; i += 1
        w2_ref = refs[i]; i += 1
        b2_ref = refs[i]; i += 1
        if with_s:
            mw1_ref = refs[i]; i += 1
            mb1_ref = refs[i]; i += 1
            mw2_ref = refs[i]; i += 1
            mb2_ref = refs[i]; i += 1
        nso_ref = refs[i]; i += 1
        nvo_ref = refs[i]; i += 1

        seg = seg_ref[...]
        ns1 = ns_ref[...] + seg[:, 0:HIDDEN]
        nv1 = seg[:, HIDDEN:4 * HIDDEN]
        if has_nv:
            nv1 = nv1 + nv_ref[...][:, 3 * HIDDEN:6 * HIDDEN]
        u = u_ref[...]
        v = v_ref[...]
        uv = []
        vv = []
        for dd in range(3):
            nv_d = nv1[:, dd * HIDDEN:(dd + 1) * HIDDEN]
            uv.append(jnp.dot(nv_d, u, preferred_element_type=jnp.float32))
            vv.append(jnp.dot(nv_d, v, preferred_element_type=jnp.float32))
        vn = jnp.sqrt(vv[0] * vv[0] + vv[1] * vv[1] + vv[2] * vv[2] + 1e-12)
        h = jnp.concatenate([ns1, vn], axis=1)
        h = jnp.dot(h, w1_ref[...], preferred_element_type=jnp.float32) + b1_ref[...]
        h = h * jax.nn.sigmoid(h)
        a = jnp.dot(h, w2_ref[...], preferred_element_type=jnp.float32) + b2_ref[...]
        a_vv = a[:, 0:HIDDEN]
        a_sv = a[:, HIDDEN:2 * HIDDEN]
        a_ss = a[:, 2 * HIDDEN:3 * HIDDEN]
        dotuv = uv[0] * vv[0] + uv[1] * vv[1] + uv[2] * vv[2]
        ns2 = ns1 + dotuv * a_sv + a_ss
        nv2 = jnp.concatenate([nv1[:, dd * HIDDEN:(dd + 1) * HIDDEN]
                               + uv[dd] * a_vv for dd in range(3)], axis=1)
        nso_ref[...] = ns2
        if with_s:
            hs = jnp.dot(ns2, mw1_ref[...], preferred_element_type=jnp.float32) + mb1_ref[...]
            hs = hs * jax.nn.sigmoid(hs)
            s_next = jnp.dot(hs, mw2_ref[...], preferred_element_type=jnp.float32) + mb2_ref[...]
            nvo_ref[...] = jnp.concatenate([s_next, nv2], axis=1)
        else:
            nvo_ref[...] = nv2

    def call(ns, nv, seg, u, v, w1, b1, w2, b2, msg_w1=None, msg_b1=None,
             msg_w2=None, msg_b2=None):
        in_specs = [pl.BlockSpec((_BN, HIDDEN), lambda i: (i, 0))]
        args = [ns]
        if has_nv:
            in_specs.append(pl.BlockSpec((_BN, 6 * HIDDEN), lambda i: (i, 0)))
            args.append(nv)
        in_specs.append(pl.BlockSpec((_BN, 4 * HIDDEN), lambda i: (i, 0)))
        args.append(seg)
        for wref in (u, v, w1, b1, w2, b2):
            in_specs.append(pl.BlockSpec(wref.shape, lambda i: (0, 0)))
            args.append(wref)
        if with_s:
            for wref in (msg_w1, msg_b1, msg_w2, msg_b2):
                in_specs.append(pl.BlockSpec(wref.shape, lambda i: (0, 0)))
                args.append(wref)
        owidth = 6 * HIDDEN if with_s else 3 * HIDDEN
        out_specs = [
            pl.BlockSpec((_BN, HIDDEN), lambda i: (i, 0)),
            pl.BlockSpec((_BN, owidth), lambda i: (i, 0)),
        ]
        out_shape = [
            jax.ShapeDtypeStruct((N_NODES, HIDDEN), jnp.float32),
            jax.ShapeDtypeStruct((N_NODES, owidth), jnp.float32),
        ]
        return pl.pallas_call(
            body,
            grid=(N_NODES // _BN,),
            in_specs=in_specs,
            out_specs=out_specs,
            out_shape=out_shape,
        )(*args)

    return call


_update_l0 = _make_update(False, True)
_update_mid = _make_update(True, True)
_update_last = _make_update(True, False)


def kernel(x_atoms, x_pos, edges, embed_table, filter_W, filter_b, msg_W1, msg_b1,
           msg_W2, msg_b2, U_W, V_W, upd_W1, upd_b1, upd_W2, upd_b2):
    src = edges[:, 0].astype(jnp.int32)
    dst = edges[:, 1].astype(jnp.int32)
    perm = jnp.argsort(dst)
    pad = E_PAD - N_EDGES
    src_p = jnp.concatenate([src[perm], jnp.arange(pad, dtype=jnp.int32) % N_NODES])
    dst_p = jnp.concatenate([dst[perm], jnp.full((pad,), N_NODES, jnp.int32)])
    n0s = jnp.arange(NRANGES, dtype=jnp.int32) * NODES_PER_R
    starts = jnp.searchsorted(dst_p, n0s).astype(jnp.int32)
    ends = jnp.searchsorted(dst_p, n0s + NODES_PER_R).astype(jnp.int32)
    rb = jnp.zeros((NRANGES, 16), jnp.int32)
    rb = rb.at[:, 0].set(starts).at[:, 1].set(ends).reshape(-1)
    xp128 = jnp.pad(x_pos.astype(jnp.float32), ((0, 0), (0, 125)))
    x_atoms2 = x_atoms.astype(jnp.int32).reshape(N_NODES, 1)

    fb2 = filter_b.reshape(NLAYERS, 1, 3 * HIDDEN)
    mb1 = msg_b1.reshape(NLAYERS, 1, HIDDEN)
    mb2 = msg_b2.reshape(NLAYERS, 1, 3 * HIDDEN)
    ub1 = upd_b1.reshape(NLAYERS, 1, HIDDEN)
    ub2 = upd_b2.reshape(NLAYERS, 1, 3 * HIDDEN)

    gps = _gather_128(xp128, src_p)
    gpd = _gather_128(xp128, jnp.minimum(dst_p, N_NODES - 1))
    geom = _geom(gps, gpd)

    ns, s = _embed_smlp(x_atoms2, embed_table, msg_W1[0], mb1[0], msg_W2[0], mb2[0])
    gt = _gather_256(s, src_p)
    rows = _msg_l0(geom, gt, filter_W[0], fb2[0])
    seg = _segsum_kernel(rows, dst_p, rb)
    ns, t = _update_l0(ns, None, seg, U_W[0], V_W[0], upd_W1[0], ub1[0],
                       upd_W2[0], ub2[0], msg_W1[1], mb1[1], msg_W2[1], mb2[1])

    gt = _gather_384(t, src_p)
    rows = _msg(geom, gt, filter_W[1], fb2[1])
    seg = _segsum_kernel(rows, dst_p, rb)
    ns, t = _update_mid(ns, t, seg, U_W[1], V_W[1], upd_W1[1], ub1[1],
                        upd_W2[1], ub2[1], msg_W1[2], mb1[2], msg_W2[2], mb2[2])

    gt = _gather_384(t, src_p)
    rows = _msg(geom, gt, filter_W[2], fb2[2])
    seg = _segsum_kernel(rows, dst_p, rb)
    ns, nv = _update_last(ns, t, seg, U_W[2], V_W[2], upd_W1[2], ub1[2],
                          upd_W2[2], ub2[2])

    return ns, nv.reshape(N_NODES, 3, HIDDEN)
